# Initial kernel scaffold; baseline (speedup 1.0000x reference)
import jax
import jax.numpy as jnp
from jax.experimental import pallas as pl


def kernel(feature, edge_index, W, b):
    raise NotImplementedError("write your pallas kernel here")


# speedup vs baseline: 5.7420x; 5.7420x over previous
import functools

import jax
import jax.numpy as jnp
from jax import lax
from jax.experimental import pallas as pl
from jax.experimental.pallas import tpu as pltpu
from jax.experimental.pallas import tpu_sc as plsc

N = 10000
E = 320000
D = 128
DA = 144

NC = 2
NS = 16
NW = NC * NS
EPW = E // NW
CHUNK = 80
NCHUNK = EPW // CHUNK
RPT = N // NS
RC = 125
NRC = RPT // RC


def _sc_aggregate(faug, src, dst):
    mesh = plsc.VectorSubcoreMesh(core_axis_name="c", subcore_axis_name="s")

    @functools.partial(
        pl.kernel,
        mesh=mesh,
        compiler_params=pltpu.CompilerParams(use_tc_tiling_on_sc=False),
        out_type=jax.ShapeDtypeStruct((NC, N, DA), jnp.float32),
        scratch_types=[
            pltpu.VMEM((CHUNK,), jnp.int32),
            pltpu.VMEM((CHUNK,), jnp.int32),
            pltpu.VMEM((CHUNK, DA), jnp.float32),
            pltpu.VMEM((RC, DA), jnp.float32),
            pltpu.VMEM_SHARED((N, DA), jnp.float32),
            pltpu.SemaphoreType.DMA,
        ],
    )
    def k(faug_hbm, src_hbm, dst_hbm, out_hbm, src_v, dst_v, rows_v, cbuf,
          acc_sh, sem):
        cid = lax.axis_index("c")
        sid = lax.axis_index("s")
        wid = sid * NC + cid

        def zrow(r, _):
            def zcol(q, _):
                cbuf[r, pl.ds(q * 16, 16)] = jnp.zeros((16,), jnp.float32)
                return _
            return lax.fori_loop(0, DA // 16, zcol, _)
        lax.fori_loop(0, RC, zrow, None)
        for j in range(NRC):
            pltpu.sync_copy(cbuf, acc_sh.at[pl.ds(sid * RPT + j * RC, RC)])
        plsc.subcore_barrier()

        ebase = wid * EPW

        def body(i, _):
            base = ebase + i * CHUNK
            pltpu.sync_copy(src_hbm.at[pl.ds(base, CHUNK)], src_v)
            pltpu.sync_copy(dst_hbm.at[pl.ds(base, CHUNK)], dst_v)
            pltpu.async_copy(faug_hbm.at[src_v], rows_v, sem).wait()
            pltpu.sync_copy(rows_v, acc_sh.at[dst_v], add=True)
            return _
        lax.fori_loop(0, NCHUNK, body, None)
        plsc.subcore_barrier()

        for j in range(NRC):
            r0 = sid * RPT + j * RC
            pltpu.sync_copy(acc_sh.at[pl.ds(r0, RC)], cbuf)
            pltpu.sync_copy(cbuf, out_hbm.at[cid, pl.ds(r0, RC)])

    return k(faug, src, dst)


def _tc_combine(feature, parts, W, b):
    def body(f_ref, p_ref, w_ref, b_ref, o_ref):
        a = p_ref[0] + p_ref[1]
        agg = a[:, :D]
        deg = jnp.sum(a[:, D:], axis=1, keepdims=True)
        neigh = agg / jnp.maximum(deg, 1.0)
        out = (
            jnp.dot(f_ref[...], w_ref[:D, :], preferred_element_type=jnp.float32)
            + jnp.dot(neigh, w_ref[D:, :], preferred_element_type=jnp.float32)
            + b_ref[...][None, :]
        )
        o_ref[...] = jnp.maximum(out, 0.0)

    return pl.pallas_call(
        body,
        out_shape=jax.ShapeDtypeStruct((N, D), jnp.float32),
    )(feature, parts, W, b)


def kernel(feature, edge_index, W, b):
    faug = jnp.concatenate(
        [feature,
         jnp.ones((N, 1), feature.dtype),
         jnp.zeros((N, DA - D - 1), feature.dtype)],
        axis=1,
    )
    src = edge_index[0]
    dst = edge_index[1]
    parts = _sc_aggregate(faug, src, dst)
    return _tc_combine(feature, parts, W, b)
